# baseline (speedup 1.0000x reference)
import jax
import jax.numpy as jnp
from jax import lax
from jax.experimental import pallas as pl
from jax.experimental.pallas import tpu as pltpu
from jax.experimental.pallas import tpu_sc as plsc

_N = 10000
_E = 320000
_DIN = 128
_H = 64
_DOUT = 128

_NC = 2
_NS = 16
_NW = _NC * _NS
_EW = _E // _NW
_CHUNK = 128
_KCH = 78
_GRP = 8
_LAST = _KCH - 9 * _GRP

_NBUF = 4


def _segsum_body(tbl_hbm, ei_hbm, zeros_hbm, out_hbm,
                 rows, srcb, dstb, tbl_sh, acc_sh,
                 gsems, ssems, isems, psem):
    c = lax.axis_index("c")
    s = lax.axis_index("s")
    wid = s * _NC + c
    cb = wid * _KCH
    rps = _N // _NS
    r0 = s * rps

    half = s % (_NS // 2)
    col0 = (s // (_NS // 2)) * _H
    trow0 = (s // (_NS // 2)) * (_N // 2) + half * rps
    pltpu.async_copy(zeros_hbm, acc_sh.at[pl.ds(r0, rps)], psem)
    pltpu.async_copy(
        tbl_hbm.at[pl.ds(half * rps, rps), pl.ds(col0, _H)],
        tbl_sh.at[pl.ds(trow0, rps)], psem)

    def iload_start(t_row, buf, n):
        pltpu.async_copy(ei_hbm.at[0, pl.ds(t_row, n)],
                         srcb[buf].at[pl.ds(0, n)], isems[buf])
        pltpu.async_copy(ei_hbm.at[1, pl.ds(t_row, n)],
                         dstb[buf].at[pl.ds(0, n)], isems[buf])

    def iload_wait(buf, n):
        pltpu.make_async_copy(ei_hbm.at[0, pl.ds(0, n)],
                              srcb[buf].at[pl.ds(0, n)], isems[buf]).wait()
        pltpu.make_async_copy(ei_hbm.at[1, pl.ds(0, n)],
                              dstb[buf].at[pl.ds(0, n)], isems[buf]).wait()

    def gather_start(buf, b, rb):
        pltpu.async_copy(tbl_sh.at[srcb[buf].at[b]], rows[rb], gsems[rb])

    def gather_wait(buf, b, rb):
        pltpu.make_async_copy(tbl_sh.at[srcb[buf].at[b]], rows[rb],
                              gsems[rb]).wait()

    def scat_start(buf, b, rb):
        pltpu.async_copy(rows[rb], acc_sh.at[dstb[buf].at[b]], ssems[rb],
                         add=True)

    def scat_wait(buf, b, rb):
        pltpu.make_async_copy(rows[rb], acc_sh.at[dstb[buf].at[b]],
                              ssems[rb]).wait()

    iload_start(cb, 0, _GRP)
    iload_start(cb + _GRP, 1, _GRP)

    pltpu.make_async_copy(zeros_hbm, acc_sh.at[pl.ds(r0, rps)], psem).wait()
    pltpu.make_async_copy(
        tbl_hbm.at[pl.ds(half * rps, rps), pl.ds(col0, _H)],
        tbl_sh.at[pl.ds(trow0, rps)], psem).wait()
    plsc.subcore_barrier()

    iload_wait(0, _GRP)
    gather_start(0, 0, 0)
    gather_start(0, 1, 1)

    def step(g, b, buf, guard_ssem=True, do_gather=True, do_iload=True,
             il_n=_GRP, wait_n=None):
        rb = b % _NBUF
        gather_wait(buf, b, rb)
        scat_start(buf, b, rb)
        if do_gather:
            nbuf_g = buf if b < _GRP - 2 else 1 - buf
            nb = (b + 2) % _GRP
            rg = (b + 2) % _NBUF
            if wait_n is not None:
                iload_wait(nbuf_g, wait_n)
            if guard_ssem:
                pbuf = buf if b >= 2 else 1 - buf
                pb = (b - 2) % _GRP
                scat_wait(pbuf, pb, rg)
            gather_start(nbuf_g, nb, rg)
        if do_iload:
            iload_start(cb + (g + 1) * _GRP, 1 - buf, il_n)

    def group(g, buf, first=False, il_n=_GRP, wait_n=_GRP):
        for b in range(_GRP):
            step(g, b, buf, guard_ssem=(not first) or b >= 2,
                 do_iload=(b == 2 and il_n is not None),
                 il_n=il_n if il_n is not None else _GRP,
                 wait_n=wait_n if b == _GRP - 2 else None)

    group(0, 0, first=True, il_n=None)
    group(1, 1)

    def body(di, carry):
        for k, buf in ((0, 0), (1, 1)):
            group(2 * di + k, buf)
        return carry

    lax.fori_loop(1, 3, body, 0)

    group(6, 0)
    group(7, 1)
    group(8, 0, il_n=_LAST, wait_n=_LAST)
    for b in range(_LAST):
        step(9, b, 1, do_gather=(b + 2 < _LAST + 2 and 72 + b + 2 < _KCH),
             do_iload=False)
    for b in range(2, _LAST):
        scat_wait(1, b, (72 + b) % _NBUF)

    @pl.when(wid < _E // _CHUNK - _NW * _KCH)
    def _tail():
        iload_start(_NW * _KCH + wid, 0, 1)
        iload_wait(0, 1)
        gather_start(0, 0, 0)
        gather_wait(0, 0, 0)
        scat_start(0, 0, 0)
        scat_wait(0, 0, 0)

    plsc.subcore_barrier()
    pltpu.sync_copy(acc_sh.at[pl.ds(r0, rps)],
                    out_hbm.at[pl.ds(r0, rps), pl.ds(c * _H, _H)])


_segsum = pl.kernel(
    _segsum_body,
    mesh=plsc.VectorSubcoreMesh(core_axis_name="c", subcore_axis_name="s"),
    out_type=jax.ShapeDtypeStruct((_N, 2 * _H), jnp.float32),
    scratch_types=[
        tuple(pltpu.VMEM((_CHUNK, _H), jnp.float32) for _ in range(_NBUF)),
        tuple(pltpu.VMEM((_GRP, _CHUNK), jnp.int32) for _ in range(2)),
        tuple(pltpu.VMEM((_GRP, _CHUNK), jnp.int32) for _ in range(2)),
        pltpu.VMEM_SHARED((_N, _H), jnp.float32),
        pltpu.VMEM_SHARED((_N, _H), jnp.float32),
        tuple(pltpu.SemaphoreType.DMA for _ in range(_NBUF)),
        tuple(pltpu.SemaphoreType.DMA for _ in range(_NBUF)),
        tuple(pltpu.SemaphoreType.DMA for _ in range(2)),
        pltpu.SemaphoreType.DMA,
    ],
    compiler_params=pltpu.CompilerParams(use_tc_tiling_on_sc=False),
)


def _bn(t, g, b):
    mean = jnp.mean(t, axis=0, keepdims=True)
    var = jnp.mean(jnp.square(t - mean), axis=0, keepdims=True)
    return (t - mean) * lax.rsqrt(var + 1e-5) * g + b


def _mm_k(x_ref, w_ref, o_ref):
    top = jnp.dot(x_ref[:_N // 2], w_ref[...],
                  preferred_element_type=jnp.float32)
    bot = jnp.dot(x_ref[_N // 2:], w_ref[...],
                  preferred_element_type=jnp.float32)
    o_ref[...] = jnp.concatenate([top, bot], axis=1)


def _stage_b_k(p_ref, part_ref, b0a_ref, g0a_ref, be0a_ref, w1a_ref, b1a_ref,
               gbn_ref, bbn_ref, w0b_ref, z_ref, q_ref):
    agg = part_ref[:, :_H] + part_ref[:, _H:]
    p = jnp.concatenate([p_ref[:, :_H], p_ref[:, _H:]], axis=0)
    t = p + agg + b0a_ref[...]
    y = jnp.maximum(_bn(t, g0a_ref[...], be0a_ref[...]), 0.0)
    z = jnp.dot(y, w1a_ref[...], preferred_element_type=jnp.float32) + b1a_ref[...]
    z_ref[...] = z
    hh = jnp.maximum(_bn(z, gbn_ref[...], bbn_ref[...]), 0.0)
    q = jnp.dot(hh, w0b_ref[...], preferred_element_type=jnp.float32)
    q_ref[...] = jnp.concatenate([q[:_N // 2], q[_N // 2:]], axis=1)


def _stage_c_k(q_ref, part_ref, b0b_ref, g0b_ref, be0b_ref, w1b_ref, b1b_ref,
               o_ref):
    agg = part_ref[:, :_H] + part_ref[:, _H:]
    q = jnp.concatenate([q_ref[:, :_H], q_ref[:, _H:]], axis=0)
    t = q + agg + b0b_ref[...]
    y = jnp.maximum(_bn(t, g0b_ref[...], be0b_ref[...]), 0.0)
    o_ref[...] = jnp.dot(y, w1b_ref[...],
                         preferred_element_type=jnp.float32) + b1b_ref[...]


def kernel(x, edge_index, W0a, b0a, g0a, be0a, W1a, b1a, g_bn1, b_bn1,
           W0b, b0b, g0b, be0b, W1b, b1b):
    zeros_sub = jnp.zeros((_N // _NS, _H), jnp.float32)

    p1 = pl.pallas_call(
        _mm_k,
        out_shape=jax.ShapeDtypeStruct((_N // 2, 2 * _H), jnp.float32),
    )(x, W0a)

    ei3 = edge_index.reshape(2, _E // _CHUNK, _CHUNK)
    part1 = _segsum(p1, ei3, zeros_sub)

    z, q = pl.pallas_call(
        _stage_b_k,
        out_shape=(jax.ShapeDtypeStruct((_N, _H), jnp.float32),
                   jax.ShapeDtypeStruct((_N // 2, 2 * _H), jnp.float32)),
    )(p1, part1, b0a.reshape(1, _H), g0a.reshape(1, _H), be0a.reshape(1, _H),
      W1a, b1a.reshape(1, _H), g_bn1.reshape(1, _H), b_bn1.reshape(1, _H),
      W0b)

    part2 = _segsum(q, ei3, zeros_sub)

    out = pl.pallas_call(
        _stage_c_k,
        out_shape=jax.ShapeDtypeStruct((_N, _DOUT), jnp.float32),
    )(q, part2, b0b.reshape(1, _H), g0b.reshape(1, _H), be0b.reshape(1, _H),
      W1b, b1b.reshape(1, _DOUT))

    return (out, z)


# speedup vs baseline: 1.0249x; 1.0249x over previous
import jax
import jax.numpy as jnp
from jax import lax
from jax.experimental import pallas as pl
from jax.experimental.pallas import tpu as pltpu
from jax.experimental.pallas import tpu_sc as plsc

_N = 10000
_E = 320000
_DIN = 128
_H = 64
_DOUT = 128

_NC = 2
_NS = 16
_NW = _NC * _NS
_EW = _E // _NW
_CHUNK = 128
_KCH = _EW // _CHUNK
_TAIL = _EW - _KCH * _CHUNK

_NBUF = 4
_NBI = 8


def _segsum_body(tbl_hbm, ei_hbm, zeros_hbm, out_hbm,
                 rows, idxs, tsrc, tdst, trow, tbl_sh, acc_sh,
                 gsems, ssems, isems, psem, tsem):
    c = lax.axis_index("c")
    s = lax.axis_index("s")
    wid = s * _NC + c
    ebase = wid * _EW
    rps = _N // _NS
    r0 = s * rps

    half = s % (_NS // 2)
    col0 = (s // (_NS // 2)) * _H
    trow0 = (s // (_NS // 2)) * (_N // 2) + half * rps
    pltpu.async_copy(zeros_hbm, acc_sh.at[pl.ds(r0, rps)], psem)
    pltpu.async_copy(
        tbl_hbm.at[pl.ds(half * rps, rps), pl.ds(col0, _H)],
        tbl_sh.at[pl.ds(trow0, rps)], psem)

    def iload_start(j, jb):
        e0 = ebase + j * _CHUNK
        pltpu.async_copy(ei_hbm.at[0, pl.ds(e0, _CHUNK)], idxs[jb].at[0],
                         isems[jb])
        pltpu.async_copy(ei_hbm.at[1, pl.ds(e0, _CHUNK)], idxs[jb].at[1],
                         isems[jb])

    def iload_wait(j, jb):
        e0 = ebase + j * _CHUNK
        pltpu.make_async_copy(ei_hbm.at[0, pl.ds(e0, _CHUNK)],
                              idxs[jb].at[0], isems[jb]).wait()
        pltpu.make_async_copy(ei_hbm.at[1, pl.ds(e0, _CHUNK)],
                              idxs[jb].at[1], isems[jb]).wait()

    def gather_start(jb, rb):
        pltpu.async_copy(tbl_sh.at[idxs[jb].at[0]], rows[rb], gsems[rb])

    def gather_wait(jb, rb):
        pltpu.make_async_copy(tbl_sh.at[idxs[jb].at[0]], rows[rb],
                              gsems[rb]).wait()

    def scat_start(jb, rb):
        pltpu.async_copy(rows[rb], acc_sh.at[idxs[jb].at[1]], ssems[rb],
                         add=True)

    def scat_wait(jb, rb):
        pltpu.make_async_copy(rows[rb], acc_sh.at[idxs[jb].at[1]],
                              ssems[rb]).wait()

    for b in range(_NBUF):
        iload_start(b, b)

    pltpu.make_async_copy(zeros_hbm, acc_sh.at[pl.ds(r0, rps)], psem).wait()
    pltpu.make_async_copy(
        tbl_hbm.at[pl.ds(half * rps, rps), pl.ds(col0, _H)],
        tbl_sh.at[pl.ds(trow0, rps)], psem).wait()
    plsc.subcore_barrier()

    def step(j, b, guard_ssem=True, do_gather=True, do_iload=True):
        rb = b % _NBUF
        gather_wait(b, rb)
        scat_start(b, rb)
        if do_gather:
            bg = (b + 2) % _NBI
            rg = (b + 2) % _NBUF
            iload_wait(j + 2, bg)
            if guard_ssem:
                scat_wait((b - 2) % _NBI, rg)
            gather_start(bg, rg)
        if do_iload:
            iload_start(j + 4, (b + 4) % _NBI)

    for b in range(2):
        iload_wait(b, b)
        gather_start(b, b)
    for b in range(_NBI):
        step(b, b, guard_ssem=(b >= 2))

    def body(g, carry):
        j0 = g * _NBI
        for b in range(_NBI):
            step(j0 + b, b)
        return carry

    lax.fori_loop(1, _KCH // _NBI, body, 0)

    for b in range(_KCH - (_KCH // _NBI) * _NBI):
        j = (_KCH // _NBI) * _NBI + b
        step(j, b, do_gather=(j + 2 < _KCH), do_iload=(j + 4 < _KCH))
    for b in range(_NBUF):
        j = _KCH - _NBUF + b
        scat_wait(j % _NBI, j % _NBUF)

    e0 = ebase + _KCH * _CHUNK
    pltpu.async_copy(ei_hbm.at[0, pl.ds(e0, _TAIL)], tsrc, tsem)
    pltpu.async_copy(ei_hbm.at[1, pl.ds(e0, _TAIL)], tdst, tsem)
    pltpu.make_async_copy(ei_hbm.at[0, pl.ds(e0, _TAIL)], tsrc, tsem).wait()
    pltpu.make_async_copy(ei_hbm.at[1, pl.ds(e0, _TAIL)], tdst, tsem).wait()
    pltpu.async_copy(tbl_sh.at[tsrc], trow, tsem)
    pltpu.make_async_copy(tbl_sh.at[tsrc], trow, tsem).wait()
    pltpu.sync_copy(trow, acc_sh.at[tdst], add=True)

    plsc.subcore_barrier()
    pltpu.sync_copy(acc_sh.at[pl.ds(r0, rps)],
                    out_hbm.at[pl.ds(r0, rps), pl.ds(c * _H, _H)])


_segsum = pl.kernel(
    _segsum_body,
    mesh=plsc.VectorSubcoreMesh(core_axis_name="c", subcore_axis_name="s"),
    out_type=jax.ShapeDtypeStruct((_N, 2 * _H), jnp.float32),
    scratch_types=[
        tuple(pltpu.VMEM((_CHUNK, _H), jnp.float32) for _ in range(_NBUF)),
        tuple(pltpu.VMEM((2, _CHUNK), jnp.int32) for _ in range(_NBI)),
        pltpu.VMEM((_TAIL,), jnp.int32),
        pltpu.VMEM((_TAIL,), jnp.int32),
        pltpu.VMEM((_TAIL, _H), jnp.float32),
        pltpu.VMEM_SHARED((_N, _H), jnp.float32),
        pltpu.VMEM_SHARED((_N, _H), jnp.float32),
        tuple(pltpu.SemaphoreType.DMA for _ in range(_NBUF)),
        tuple(pltpu.SemaphoreType.DMA for _ in range(_NBUF)),
        tuple(pltpu.SemaphoreType.DMA for _ in range(_NBI)),
        pltpu.SemaphoreType.DMA,
        pltpu.SemaphoreType.DMA,
    ],
    compiler_params=pltpu.CompilerParams(use_tc_tiling_on_sc=False),
)


def _bn(t, g, b):
    mean = jnp.mean(t, axis=0, keepdims=True)
    msq = jnp.mean(jnp.square(t), axis=0, keepdims=True)
    var = msq - jnp.square(mean)
    return (t - mean) * lax.rsqrt(var + 1e-5) * g + b


def _mm_k(x_ref, w_ref, o_ref):
    top = jnp.dot(x_ref[:_N // 2], w_ref[...],
                  preferred_element_type=jnp.float32)
    bot = jnp.dot(x_ref[_N // 2:], w_ref[...],
                  preferred_element_type=jnp.float32)
    o_ref[...] = jnp.concatenate([top, bot], axis=1)


def _stage_b_k(p_ref, part_ref, b0a_ref, g0a_ref, be0a_ref, w1a_ref, b1a_ref,
               gbn_ref, bbn_ref, w0b_ref, z_ref, q_ref):
    agg = part_ref[:, :_H] + part_ref[:, _H:]
    p = jnp.concatenate([p_ref[:, :_H], p_ref[:, _H:]], axis=0)
    t = p + agg + b0a_ref[...]
    y = jnp.maximum(_bn(t, g0a_ref[...], be0a_ref[...]), 0.0)
    z = jnp.dot(y, w1a_ref[...], preferred_element_type=jnp.float32) + b1a_ref[...]
    z_ref[...] = z
    hh = jnp.maximum(_bn(z, gbn_ref[...], bbn_ref[...]), 0.0)
    q = jnp.dot(hh, w0b_ref[...], preferred_element_type=jnp.float32)
    q_ref[...] = jnp.concatenate([q[:_N // 2], q[_N // 2:]], axis=1)


def _stage_c_k(q_ref, part_ref, b0b_ref, g0b_ref, be0b_ref, w1b_ref, b1b_ref,
               o_ref):
    agg = part_ref[:, :_H] + part_ref[:, _H:]
    q = jnp.concatenate([q_ref[:, :_H], q_ref[:, _H:]], axis=0)
    t = q + agg + b0b_ref[...]
    y = jnp.maximum(_bn(t, g0b_ref[...], be0b_ref[...]), 0.0)
    o_ref[...] = jnp.dot(y, w1b_ref[...],
                         preferred_element_type=jnp.float32) + b1b_ref[...]


def kernel(x, edge_index, W0a, b0a, g0a, be0a, W1a, b1a, g_bn1, b_bn1,
           W0b, b0b, g0b, be0b, W1b, b1b):
    zeros_sub = jnp.zeros((_N // _NS, _H), jnp.float32)

    p1 = pl.pallas_call(
        _mm_k,
        out_shape=jax.ShapeDtypeStruct((_N // 2, 2 * _H), jnp.float32),
    )(x, W0a)

    part1 = _segsum(p1, edge_index, zeros_sub)

    z, q = pl.pallas_call(
        _stage_b_k,
        out_shape=(jax.ShapeDtypeStruct((_N, _H), jnp.float32),
                   jax.ShapeDtypeStruct((_N // 2, 2 * _H), jnp.float32)),
    )(p1, part1, b0a.reshape(1, _H), g0a.reshape(1, _H), be0a.reshape(1, _H),
      W1a, b1a.reshape(1, _H), g_bn1.reshape(1, _H), b_bn1.reshape(1, _H),
      W0b)

    part2 = _segsum(q, edge_index, zeros_sub)

    out = pl.pallas_call(
        _stage_c_k,
        out_shape=jax.ShapeDtypeStruct((_N, _DOUT), jnp.float32),
    )(q, part2, b0b.reshape(1, _H), g0b.reshape(1, _H), be0b.reshape(1, _H),
      W1b, b1b.reshape(1, _DOUT))

    return (out, z)
